# baseline (speedup 1.0000x reference)
import numpy as np

import jax
import jax.numpy as jnp
from jax import lax
from jax.experimental import pallas as pl
from jax.experimental.pallas import tpu as pltpu
from jax.experimental.pallas import tpu_sc as plsc

N_RAY = 92160
N_INT = 128
N_ROW = 512
N_COL = 512

NC = 2
NS = 16
LANES = 16
NW = NC * NS
RPW = N_RAY // NW
CHUNK = 192
NCHUNK = RPW // CHUNK
G = CHUNK // LANES
SLOTS = G * N_INT
NROWS = SLOTS * LANES // 128
UNROLL = 8

MAGIC = np.float32(12582912.0)


def _rsqrt(u):
    i = lax.bitcast_convert_type(u, jnp.int32)
    i = np.int32(0x5F3759DF) - lax.shift_right_logical(i, 1)
    y = lax.bitcast_convert_type(i, jnp.float32)
    half = np.float32(0.5) * u
    for _ in range(3):
        y = y * (np.float32(1.5) - half * y * y)
    return y


def _sc_body(t_hbm, img_hbm, sx_hbm, sy_hbm, ex_hbm, ey_hbm, scal_hbm,
             out_hbm, t_v, idx_v, vals_v, w_v, sx_v, sy_v, ex_v, ey_v,
             scal_v, out_v, sem):
    wid = lax.axis_index("s") * NC + lax.axis_index("c")
    wbase = wid * RPW
    pltpu.sync_copy(scal_hbm, scal_v)
    a00 = scal_v[0]
    a01 = scal_v[1]
    a10 = scal_v[2]
    a11 = scal_v[3]
    b0 = scal_v[4]
    b1 = scal_v[5]
    iota = lax.iota(jnp.int32, LANES)
    zeros_i = jnp.zeros((LANES,), jnp.int32)
    zeros_f = jnp.zeros((LANES,), jnp.float32)

    def chunk_body(k, _):
        base = wbase + k * CHUNK
        pltpu.sync_copy(t_hbm.at[pl.ds(base, CHUNK), :], t_v)
        pltpu.sync_copy(sx_hbm.at[pl.ds(base, CHUNK)], sx_v)
        pltpu.sync_copy(sy_hbm.at[pl.ds(base, CHUNK)], sy_v)
        pltpu.sync_copy(ex_hbm.at[pl.ds(base, CHUNK)], ex_v)
        pltpu.sync_copy(ey_hbm.at[pl.ds(base, CHUNK)], ey_v)

        def group_body(g, _):
            rows16 = g * LANES + iota
            sx = sx_v[pl.ds(g * LANES, LANES)]
            sy = sy_v[pl.ds(g * LANES, LANES)]
            dx = ex_v[pl.ds(g * LANES, LANES)] - sx
            dy = ey_v[pl.ds(g * LANES, LANES)] - sy
            u = dx * dx + dy * dy
            length = u * _rsqrt(u)
            @plsc.parallel_loop(0, N_INT - 1, unroll=UNROLL)
            def _(i):
                ci = jnp.full((LANES,), i, jnp.int32)
                cn = jnp.full((LANES,), i + 1, jnp.int32)
                tc = plsc.load_gather(t_v, [rows16, ci])
                tn = plsc.load_gather(t_v, [rows16, cn])
                xc = sx + tc * dx
                yc = sy + tc * dy
                xn = sx + tn * dx
                yn = sy + tn * dy
                mx = np.float32(0.5) * (xc + xn)
                my = np.float32(0.5) * (yc + yn)
                mxs = mx - b0
                mys = my - b1
                rowf = a00 * mxs + a01 * mys
                colf = a10 * mxs + a11 * mys
                rr = (rowf + MAGIC) - MAGIC
                cc = (colf + MAGIC) - MAGIC
                w = (tn - tc) * length
                valid = ((rr >= np.float32(0.0)) & (rr <= np.float32(511.0))
                         & (cc >= np.float32(0.0))
                         & (cc <= np.float32(511.0)))
                flatf = rr * np.float32(N_COL) + cc
                flatf = jnp.where(valid, flatf, np.float32(0.0))
                w = jnp.where(valid, w, np.float32(0.0))
                idx = flatf.astype(jnp.int32)
                s = g * N_INT + i
                brow = s >> 3
                bcol = (s & 7) * LANES
                idx_v[brow, pl.ds(bcol, LANES)] = idx
                w_v[pl.ds(s * LANES, LANES)] = w

            idx_v[g * LANES + 15, pl.ds(112, LANES)] = zeros_i
            w_v[pl.ds((g * N_INT + N_INT - 1) * LANES, LANES)] = zeros_f
            return 0

        lax.fori_loop(0, G, group_body, 0)

        def fire(jb, _):
            for u in range(UNROLL):
                j = jb * UNROLL + u
                pltpu.async_copy(img_hbm.at[idx_v.at[j]], vals_v.at[j], sem)
            return 0

        lax.fori_loop(0, NROWS // UNROLL, fire, 0)
        pltpu.make_async_copy(t_hbm.at[pl.ds(0, CHUNK), :], vals_v,
                              sem).wait()

        def acc_group(g, _):
            def acc_block(ib, acc):
                accs = list(acc)
                for u in range(UNROLL):
                    s = g * N_INT + ib * UNROLL + u
                    brow = s >> 3
                    bcol = (s & 7) * LANES
                    v = vals_v[brow, pl.ds(bcol, LANES)]
                    wv = w_v[pl.ds(s * LANES, LANES)]
                    accs[u % 4] = accs[u % 4] + v * wv
                return tuple(accs)

            acc = lax.fori_loop(0, N_INT // UNROLL, acc_block,
                                (zeros_f,) * 4)
            out_v[pl.ds(g * LANES, LANES)] = ((acc[0] + acc[1])
                                              + (acc[2] + acc[3]))
            return 0

        lax.fori_loop(0, G, acc_group, 0)
        pltpu.sync_copy(out_v, out_hbm.at[pl.ds(base, CHUNK)])
        return 0

    lax.fori_loop(0, NCHUNK, chunk_body, 0)


@jax.jit
def kernel(image, t_sorted, M, b, src, dst):
    M_inv = jnp.linalg.inv(M)
    scal = jnp.stack([
        jnp.broadcast_to(M_inv[0, 0], (LANES,)),
        jnp.broadcast_to(M_inv[0, 1], (LANES,)),
        jnp.broadcast_to(M_inv[1, 0], (LANES,)),
        jnp.broadcast_to(M_inv[1, 1], (LANES,)),
        jnp.broadcast_to(b[0], (LANES,)),
        jnp.broadcast_to(b[1], (LANES,)),
    ]).astype(jnp.float32)
    img_flat = image.reshape(-1)
    sx = src[:, 0]
    sy = src[:, 1]
    ex = dst[:, 0]
    ey = dst[:, 1]

    mesh = plsc.VectorSubcoreMesh(core_axis_name="c", subcore_axis_name="s")
    run = pl.kernel(
        _sc_body,
        out_type=jax.ShapeDtypeStruct((N_RAY,), jnp.float32),
        mesh=mesh,
        compiler_params=pltpu.CompilerParams(needs_layout_passes=False),
        scratch_types=[
            pltpu.VMEM((CHUNK, N_INT), jnp.float32),
            pltpu.VMEM((NROWS, 128), jnp.int32),
            pltpu.VMEM((NROWS, 128), jnp.float32),
            pltpu.VMEM((SLOTS * LANES,), jnp.float32),
            pltpu.VMEM((CHUNK,), jnp.float32),
            pltpu.VMEM((CHUNK,), jnp.float32),
            pltpu.VMEM((CHUNK,), jnp.float32),
            pltpu.VMEM((CHUNK,), jnp.float32),
            pltpu.VMEM((8, LANES), jnp.float32),
            pltpu.VMEM((CHUNK,), jnp.float32),
            pltpu.SemaphoreType.DMA,
        ],
    )
    return run(t_sorted, img_flat, sx, sy, ex, ey,
               jnp.pad(scal, ((0, 2), (0, 0))))


# speedup vs baseline: 1.0957x; 1.0957x over previous
import numpy as np

import jax
import jax.numpy as jnp
from jax import lax
from jax.experimental import pallas as pl
from jax.experimental.pallas import tpu as pltpu
from jax.experimental.pallas import tpu_sc as plsc

N_RAY = 92160
N_INT = 128
N_ROW = 512
N_COL = 512

NC = 2
NS = 16
LANES = 16
NW = NC * NS
RPW = N_RAY // NW
CHUNK = 192
NCHUNK = RPW // CHUNK
G = CHUNK // LANES
SLOTS = G * N_INT
NROWS = SLOTS * LANES // 128
UNROLL = 8

MAGIC = np.float32(12582912.0)
IMAGIC = np.int32(0x4B400000)


def _rsqrt(u):
    i = lax.bitcast_convert_type(u, jnp.int32)
    i = np.int32(0x5F3759DF) - lax.shift_right_logical(i, 1)
    y = lax.bitcast_convert_type(i, jnp.float32)
    half = np.float32(0.5) * u
    for _ in range(3):
        y = y * (np.float32(1.5) - half * y * y)
    return y


def _sc_body(t_hbm, img_hbm, sx_hbm, sy_hbm, ex_hbm, ey_hbm, scal_hbm,
             out_hbm, t_v, idx_v, vals_v, w_v, sx_v, sy_v, ex_v, ey_v,
             scal_v, out_v, sem):
    wid = lax.axis_index("s") * NC + lax.axis_index("c")
    wbase = wid * RPW
    pltpu.sync_copy(scal_hbm, scal_v)
    a00 = scal_v[0]
    a01 = scal_v[1]
    a10 = scal_v[2]
    a11 = scal_v[3]
    b0 = scal_v[4]
    b1 = scal_v[5]
    iota = lax.iota(jnp.int32, LANES)
    zeros_i = jnp.zeros((LANES,), jnp.int32)
    zeros_f = jnp.zeros((LANES,), jnp.float32)

    def chunk_body(k, _):
        base = wbase + k * CHUNK
        pltpu.sync_copy(t_hbm.at[pl.ds(base * N_INT, CHUNK * N_INT)], t_v)
        pltpu.sync_copy(sx_hbm.at[pl.ds(base, CHUNK)], sx_v)
        pltpu.sync_copy(sy_hbm.at[pl.ds(base, CHUNK)], sy_v)
        pltpu.sync_copy(ex_hbm.at[pl.ds(base, CHUNK)], ex_v)
        pltpu.sync_copy(ey_hbm.at[pl.ds(base, CHUNK)], ey_v)

        def group_body(g, _):
            sx = sx_v[pl.ds(g * LANES, LANES)]
            sy = sy_v[pl.ds(g * LANES, LANES)]
            dx = ex_v[pl.ds(g * LANES, LANES)] - sx
            dy = ey_v[pl.ds(g * LANES, LANES)] - sy
            u = dx * dx + dy * dy
            length = u * _rsqrt(u)
            ivec0 = (g * LANES + iota) * N_INT
            t0 = plsc.load_gather(t_v, [ivec0])
            x0 = sx + t0 * dx
            y0 = sy + t0 * dy

            @plsc.parallel_loop(0, N_INT - 1, unroll=UNROLL,
                                carry=(ivec0, t0, x0, y0))
            def _(i, carry):
                ivec, tc, xc, yc = carry
                ivn = ivec + 1
                tn = plsc.load_gather(t_v, [ivn])
                xn = sx + tn * dx
                yn = sy + tn * dy
                mx = np.float32(0.5) * (xc + xn)
                my = np.float32(0.5) * (yc + yn)
                mxs = mx - b0
                mys = my - b1
                rowf = a00 * mxs + a01 * mys
                colf = a10 * mxs + a11 * mys
                ri = lax.bitcast_convert_type(rowf + MAGIC, jnp.int32) - IMAGIC
                ci = lax.bitcast_convert_type(colf + MAGIC, jnp.int32) - IMAGIC
                valid = (lax.bitcast_convert_type(ri | ci, jnp.uint32)
                         < np.uint32(N_COL))
                flat = (ri << 9) | ci
                w = (tn - tc) * length
                idx = jnp.where(valid, flat, 0)
                w = jnp.where(valid, w, np.float32(0.0))
                s = g * N_INT + i
                brow = s >> 3
                bcol = (s & 7) * LANES
                idx_v[brow, pl.ds(bcol, LANES)] = idx
                w_v[pl.ds(s * LANES, LANES)] = w
                return ivn, tn, xn, yn

            idx_v[g * LANES + 15, pl.ds(112, LANES)] = zeros_i
            w_v[pl.ds((g * N_INT + N_INT - 1) * LANES, LANES)] = zeros_f
            return 0

        lax.fori_loop(0, G, group_body, 0)

        def fire(jb, _):
            for u in range(UNROLL):
                j = jb * UNROLL + u
                pltpu.async_copy(img_hbm.at[idx_v.at[j]],
                                 vals_v.at[pl.ds(j * 128, 128)], sem)
            return 0

        lax.fori_loop(0, NROWS // UNROLL, fire, 0)
        pltpu.make_async_copy(img_hbm.at[pl.ds(0, SLOTS * LANES)], vals_v,
                              sem).wait()

        def acc_group(g, _):
            def acc_block(ib, acc):
                accs = list(acc)
                for u in range(UNROLL):
                    s = g * N_INT + ib * UNROLL + u
                    brow = s >> 3
                    bcol = (s & 7) * LANES
                    v = vals_v[pl.ds(s * LANES, LANES)]
                    wv = w_v[pl.ds(s * LANES, LANES)]
                    accs[u % 4] = accs[u % 4] + v * wv
                return tuple(accs)

            acc = lax.fori_loop(0, N_INT // UNROLL, acc_block,
                                (zeros_f,) * 4)
            out_v[pl.ds(g * LANES, LANES)] = ((acc[0] + acc[1])
                                              + (acc[2] + acc[3]))
            return 0

        lax.fori_loop(0, G, acc_group, 0)
        pltpu.sync_copy(out_v, out_hbm.at[pl.ds(base, CHUNK)])
        return 0

    lax.fori_loop(0, NCHUNK, chunk_body, 0)


@jax.jit
def kernel(image, t_sorted, M, b, src, dst):
    M_inv = jnp.linalg.inv(M)
    scal = jnp.stack([
        jnp.broadcast_to(M_inv[0, 0], (LANES,)),
        jnp.broadcast_to(M_inv[0, 1], (LANES,)),
        jnp.broadcast_to(M_inv[1, 0], (LANES,)),
        jnp.broadcast_to(M_inv[1, 1], (LANES,)),
        jnp.broadcast_to(b[0], (LANES,)),
        jnp.broadcast_to(b[1], (LANES,)),
    ]).astype(jnp.float32)
    img_flat = image.reshape(-1)
    sx = src[:, 0]
    sy = src[:, 1]
    ex = dst[:, 0]
    ey = dst[:, 1]

    mesh = plsc.VectorSubcoreMesh(core_axis_name="c", subcore_axis_name="s")
    run = pl.kernel(
        _sc_body,
        out_type=jax.ShapeDtypeStruct((N_RAY,), jnp.float32),
        mesh=mesh,
        compiler_params=pltpu.CompilerParams(needs_layout_passes=False),
        scratch_types=[
            pltpu.VMEM((CHUNK * N_INT,), jnp.float32),
            pltpu.VMEM((NROWS, 128), jnp.int32),
            pltpu.VMEM((SLOTS * LANES,), jnp.float32),
            pltpu.VMEM((SLOTS * LANES,), jnp.float32),
            pltpu.VMEM((CHUNK,), jnp.float32),
            pltpu.VMEM((CHUNK,), jnp.float32),
            pltpu.VMEM((CHUNK,), jnp.float32),
            pltpu.VMEM((CHUNK,), jnp.float32),
            pltpu.VMEM((8, LANES), jnp.float32),
            pltpu.VMEM((CHUNK,), jnp.float32),
            pltpu.SemaphoreType.DMA,
        ],
    )
    return run(t_sorted.reshape(-1), img_flat, sx, sy, ex, ey,
               jnp.pad(scal, ((0, 2), (0, 0))))


# speedup vs baseline: 1.6112x; 1.4704x over previous
import numpy as np

import jax
import jax.numpy as jnp
from jax import lax
from jax.experimental import pallas as pl
from jax.experimental.pallas import tpu as pltpu
from jax.experimental.pallas import tpu_sc as plsc

N_RAY = 92160
N_INT = 128
N_ROW = 512
N_COL = 512

NC = 2
NS = 16
LANES = 16
NW = NC * NS
RPW = N_RAY // NW
CHUNK = 192
NCHUNK = RPW // CHUNK
G = CHUNK // LANES
SLOTS = G * N_INT
NROWS = SLOTS * LANES // 128
UNROLL = 8

MAGIC = np.float32(12582912.0)
IMAGIC = np.int32(0x4B400000)


def _rsqrt(u):
    i = lax.bitcast_convert_type(u, jnp.int32)
    i = np.int32(0x5F3759DF) - lax.shift_right_logical(i, 1)
    y = lax.bitcast_convert_type(i, jnp.float32)
    half = np.float32(0.5) * u
    for _ in range(3):
        y = y * (np.float32(1.5) - half * y * y)
    return y


def _sc_body(t_hbm, img_hbm, sx_hbm, sy_hbm, ex_hbm, ey_hbm, scal_hbm,
             out_hbm, t_v, idx_v, vals_v, w_v, sx_v, sy_v, ex_v, ey_v,
             scal_v, out_v, sem):
    wid = lax.axis_index("s") * NC + lax.axis_index("c")
    wbase = wid * RPW
    pltpu.sync_copy(scal_hbm, scal_v)
    a00 = scal_v[0]
    a01 = scal_v[1]
    a10 = scal_v[2]
    a11 = scal_v[3]
    b0 = scal_v[4]
    b1 = scal_v[5]
    iota = lax.iota(jnp.int32, LANES)
    zeros_i = jnp.zeros((LANES,), jnp.int32)
    zeros_f = jnp.zeros((LANES,), jnp.float32)

    def chunk_body(k, _):
        base = wbase + k * CHUNK
        pltpu.sync_copy(t_hbm.at[pl.ds(base * N_INT, CHUNK * N_INT)], t_v)
        pltpu.sync_copy(sx_hbm.at[pl.ds(base, CHUNK)], sx_v)
        pltpu.sync_copy(sy_hbm.at[pl.ds(base, CHUNK)], sy_v)
        pltpu.sync_copy(ex_hbm.at[pl.ds(base, CHUNK)], ex_v)
        pltpu.sync_copy(ey_hbm.at[pl.ds(base, CHUNK)], ey_v)

        def group_body(g, _):
            sx = sx_v[pl.ds(g * LANES, LANES)]
            sy = sy_v[pl.ds(g * LANES, LANES)]
            dx = ex_v[pl.ds(g * LANES, LANES)] - sx
            dy = ey_v[pl.ds(g * LANES, LANES)] - sy
            u = dx * dx + dy * dy
            length = u * _rsqrt(u)
            ivec0 = (g * LANES + iota) * N_INT
            t0 = plsc.load_gather(t_v, [ivec0])
            x0 = sx + t0 * dx
            y0 = sy + t0 * dy

            @plsc.parallel_loop(0, N_INT - 1, unroll=UNROLL,
                                carry=(ivec0, t0, x0, y0))
            def _(i, carry):
                ivec, tc, xc, yc = carry
                ivn = ivec + 1
                tn = plsc.load_gather(t_v, [ivn])
                xn = sx + tn * dx
                yn = sy + tn * dy
                mx = np.float32(0.5) * (xc + xn)
                my = np.float32(0.5) * (yc + yn)
                mxs = mx - b0
                mys = my - b1
                rowf = a00 * mxs + a01 * mys
                colf = a10 * mxs + a11 * mys
                ri = lax.bitcast_convert_type(rowf + MAGIC, jnp.int32) - IMAGIC
                ci = lax.bitcast_convert_type(colf + MAGIC, jnp.int32) - IMAGIC
                valid = (lax.bitcast_convert_type(ri | ci, jnp.uint32)
                         < np.uint32(N_COL))
                flat = (ri << 9) | ci
                w = (tn - tc) * length
                idx = jnp.where(valid, flat, 0)
                w = jnp.where(valid, w, np.float32(0.0))
                s = g * N_INT + i
                brow = s >> 3
                bcol = (s & 7) * LANES
                idx_v[brow, pl.ds(bcol, LANES)] = idx
                w_v[pl.ds(s * LANES, LANES)] = w
                return ivn, tn, xn, yn

            idx_v[g * LANES + 15, pl.ds(112, LANES)] = zeros_i
            w_v[pl.ds((g * N_INT + N_INT - 1) * LANES, LANES)] = zeros_f
            for r in range(LANES):
                j = g * LANES + r
                pltpu.async_copy(img_hbm.at[idx_v.at[j]],
                                 vals_v.at[pl.ds(j * 128, 128)], sem)
            return 0

        lax.fori_loop(0, G, group_body, 0)

        pltpu.make_async_copy(img_hbm.at[pl.ds(0, SLOTS * LANES)], vals_v,
                              sem).wait()

        def acc_group(g, _):
            def acc_block(ib, acc):
                accs = list(acc)
                for u in range(UNROLL):
                    s = g * N_INT + ib * UNROLL + u
                    brow = s >> 3
                    bcol = (s & 7) * LANES
                    v = vals_v[pl.ds(s * LANES, LANES)]
                    wv = w_v[pl.ds(s * LANES, LANES)]
                    accs[u % 4] = accs[u % 4] + v * wv
                return tuple(accs)

            acc = lax.fori_loop(0, N_INT // UNROLL, acc_block,
                                (zeros_f,) * 4)
            out_v[pl.ds(g * LANES, LANES)] = ((acc[0] + acc[1])
                                              + (acc[2] + acc[3]))
            return 0

        lax.fori_loop(0, G, acc_group, 0)
        pltpu.sync_copy(out_v, out_hbm.at[pl.ds(base, CHUNK)])
        return 0

    lax.fori_loop(0, NCHUNK, chunk_body, 0)


@jax.jit
def kernel(image, t_sorted, M, b, src, dst):
    M_inv = jnp.linalg.inv(M)
    scal = jnp.stack([
        jnp.broadcast_to(M_inv[0, 0], (LANES,)),
        jnp.broadcast_to(M_inv[0, 1], (LANES,)),
        jnp.broadcast_to(M_inv[1, 0], (LANES,)),
        jnp.broadcast_to(M_inv[1, 1], (LANES,)),
        jnp.broadcast_to(b[0], (LANES,)),
        jnp.broadcast_to(b[1], (LANES,)),
    ]).astype(jnp.float32)
    img_flat = image.reshape(-1)
    sx = src[:, 0]
    sy = src[:, 1]
    ex = dst[:, 0]
    ey = dst[:, 1]

    mesh = plsc.VectorSubcoreMesh(core_axis_name="c", subcore_axis_name="s")
    run = pl.kernel(
        _sc_body,
        out_type=jax.ShapeDtypeStruct((N_RAY,), jnp.float32),
        mesh=mesh,
        compiler_params=pltpu.CompilerParams(needs_layout_passes=False),
        scratch_types=[
            pltpu.VMEM((CHUNK * N_INT,), jnp.float32),
            pltpu.VMEM((NROWS, 128), jnp.int32),
            pltpu.VMEM((SLOTS * LANES,), jnp.float32),
            pltpu.VMEM((SLOTS * LANES,), jnp.float32),
            pltpu.VMEM((CHUNK,), jnp.float32),
            pltpu.VMEM((CHUNK,), jnp.float32),
            pltpu.VMEM((CHUNK,), jnp.float32),
            pltpu.VMEM((CHUNK,), jnp.float32),
            pltpu.VMEM((8, LANES), jnp.float32),
            pltpu.VMEM((CHUNK,), jnp.float32),
            pltpu.SemaphoreType.DMA,
        ],
    )
    return run(t_sorted.reshape(-1), img_flat, sx, sy, ex, ey,
               jnp.pad(scal, ((0, 2), (0, 0))))


# speedup vs baseline: 1.7552x; 1.0894x over previous
import numpy as np

import jax
import jax.numpy as jnp
from jax import lax
from jax.experimental import pallas as pl
from jax.experimental.pallas import tpu as pltpu
from jax.experimental.pallas import tpu_sc as plsc

N_RAY = 92160
N_INT = 128
N_ROW = 512
N_COL = 512

NC = 2
NS = 16
LANES = 16
NW = NC * NS
RPW = N_RAY // NW
CHUNK = 96
NCHUNK = RPW // CHUNK
PAIRS = NCHUNK // 2
G = CHUNK // LANES
SLOTS = G * N_INT
NROWS = SLOTS * LANES // 128
UNROLL = 8

MAGIC = np.float32(12582912.0)
IMAGIC = np.int32(0x4B400000)


def _rsqrt(u):
    i = lax.bitcast_convert_type(u, jnp.int32)
    i = np.int32(0x5F3759DF) - lax.shift_right_logical(i, 1)
    y = lax.bitcast_convert_type(i, jnp.float32)
    half = np.float32(0.5) * u
    for _ in range(3):
        y = y * (np.float32(1.5) - half * y * y)
    return y


def _sc_body(t_hbm, img_hbm, sx_hbm, sy_hbm, ex_hbm, ey_hbm, scal_hbm,
             out_hbm, t_v, idx_a, idx_b, vals_a, vals_b, w_a, w_b,
             sx_v, sy_v, ex_v, ey_v, scal_v, out_v, sem_a, sem_b):
    wid = lax.axis_index("s") * NC + lax.axis_index("c")
    wbase = wid * RPW
    pltpu.sync_copy(scal_hbm, scal_v)
    a00 = scal_v[0]
    a01 = scal_v[1]
    a10 = scal_v[2]
    a11 = scal_v[3]
    b0 = scal_v[4]
    b1 = scal_v[5]
    iota = lax.iota(jnp.int32, LANES)
    zeros_i = jnp.zeros((LANES,), jnp.int32)
    zeros_f = jnp.zeros((LANES,), jnp.float32)

    def stage(c):
        base = wbase + c * CHUNK
        pltpu.sync_copy(t_hbm.at[pl.ds(base * N_INT, CHUNK * N_INT)], t_v)
        pltpu.sync_copy(sx_hbm.at[pl.ds(base, CHUNK)], sx_v)
        pltpu.sync_copy(sy_hbm.at[pl.ds(base, CHUNK)], sy_v)
        pltpu.sync_copy(ex_hbm.at[pl.ds(base, CHUNK)], ex_v)
        pltpu.sync_copy(ey_hbm.at[pl.ds(base, CHUNK)], ey_v)

    def pass1(idx_v, vals_v, w_v, sem):
        def group_body(g, _):
            sx = sx_v[pl.ds(g * LANES, LANES)]
            sy = sy_v[pl.ds(g * LANES, LANES)]
            dx = ex_v[pl.ds(g * LANES, LANES)] - sx
            dy = ey_v[pl.ds(g * LANES, LANES)] - sy
            u = dx * dx + dy * dy
            length = u * _rsqrt(u)
            ivec0 = (g * LANES + iota) * N_INT
            t0 = plsc.load_gather(t_v, [ivec0])
            x0 = sx + t0 * dx
            y0 = sy + t0 * dy

            @plsc.parallel_loop(0, N_INT - 1, unroll=UNROLL,
                                carry=(ivec0, t0, x0, y0))
            def _(i, carry):
                ivec, tc, xc, yc = carry
                ivn = ivec + 1
                tn = plsc.load_gather(t_v, [ivn])
                xn = sx + tn * dx
                yn = sy + tn * dy
                mx = np.float32(0.5) * (xc + xn)
                my = np.float32(0.5) * (yc + yn)
                mxs = mx - b0
                mys = my - b1
                rowf = a00 * mxs + a01 * mys
                colf = a10 * mxs + a11 * mys
                ri = lax.bitcast_convert_type(rowf + MAGIC,
                                              jnp.int32) - IMAGIC
                ci = lax.bitcast_convert_type(colf + MAGIC,
                                              jnp.int32) - IMAGIC
                valid = (lax.bitcast_convert_type(ri | ci, jnp.uint32)
                         < np.uint32(N_COL))
                flat = (ri << 9) | ci
                w = (tn - tc) * length
                idx = jnp.where(valid, flat, 0)
                w = jnp.where(valid, w, np.float32(0.0))
                s = g * N_INT + i
                idx_v[s >> 3, pl.ds((s & 7) * LANES, LANES)] = idx
                w_v[pl.ds(s * LANES, LANES)] = w
                return ivn, tn, xn, yn

            idx_v[g * LANES + 15, pl.ds(112, LANES)] = zeros_i
            w_v[pl.ds((g * N_INT + N_INT - 1) * LANES, LANES)] = zeros_f
            for r in range(LANES):
                j = g * LANES + r
                pltpu.async_copy(img_hbm.at[idx_v.at[j]],
                                 vals_v.at[pl.ds(j * 128, 128)], sem)
            return 0

        lax.fori_loop(0, G, group_body, 0)

    def drain(vals_v, sem):
        pltpu.make_async_copy(img_hbm.at[pl.ds(0, SLOTS * LANES)], vals_v,
                              sem).wait()

    def pass3(c, vals_v, w_v):
        def acc_group(g, _):
            def acc_block(ib, acc):
                accs = list(acc)
                for u in range(UNROLL):
                    s = g * N_INT + ib * UNROLL + u
                    v = vals_v[pl.ds(s * LANES, LANES)]
                    wv = w_v[pl.ds(s * LANES, LANES)]
                    accs[u % 4] = accs[u % 4] + v * wv
                return tuple(accs)

            acc = lax.fori_loop(0, N_INT // UNROLL, acc_block,
                                (zeros_f,) * 4)
            out_v[pl.ds(g * LANES, LANES)] = ((acc[0] + acc[1])
                                              + (acc[2] + acc[3]))
            return 0

        lax.fori_loop(0, G, acc_group, 0)
        pltpu.sync_copy(out_v, out_hbm.at[pl.ds(wbase + c * CHUNK, CHUNK)])

    stage(0)
    pass1(idx_a, vals_a, w_a, sem_a)

    def pair_body(kk, _):
        cb = 2 * kk + 1
        stage(cb)
        pass1(idx_b, vals_b, w_b, sem_b)
        drain(vals_a, sem_a)
        pass3(cb - 1, vals_a, w_a)

        @pl.when(kk < PAIRS - 1)
        def _():
            stage(cb + 1)
            pass1(idx_a, vals_a, w_a, sem_a)

        drain(vals_b, sem_b)
        pass3(cb, vals_b, w_b)
        return 0

    lax.fori_loop(0, PAIRS, pair_body, 0)


@jax.jit
def kernel(image, t_sorted, M, b, src, dst):
    M_inv = jnp.linalg.inv(M)
    scal = jnp.stack([
        jnp.broadcast_to(M_inv[0, 0], (LANES,)),
        jnp.broadcast_to(M_inv[0, 1], (LANES,)),
        jnp.broadcast_to(M_inv[1, 0], (LANES,)),
        jnp.broadcast_to(M_inv[1, 1], (LANES,)),
        jnp.broadcast_to(b[0], (LANES,)),
        jnp.broadcast_to(b[1], (LANES,)),
    ]).astype(jnp.float32)
    img_flat = image.reshape(-1)
    sx = src[:, 0]
    sy = src[:, 1]
    ex = dst[:, 0]
    ey = dst[:, 1]

    mesh = plsc.VectorSubcoreMesh(core_axis_name="c", subcore_axis_name="s")
    run = pl.kernel(
        _sc_body,
        out_type=jax.ShapeDtypeStruct((N_RAY,), jnp.float32),
        mesh=mesh,
        compiler_params=pltpu.CompilerParams(needs_layout_passes=False),
        scratch_types=[
            pltpu.VMEM((CHUNK * N_INT,), jnp.float32),
            pltpu.VMEM((NROWS, 128), jnp.int32),
            pltpu.VMEM((NROWS, 128), jnp.int32),
            pltpu.VMEM((SLOTS * LANES,), jnp.float32),
            pltpu.VMEM((SLOTS * LANES,), jnp.float32),
            pltpu.VMEM((SLOTS * LANES,), jnp.float32),
            pltpu.VMEM((SLOTS * LANES,), jnp.float32),
            pltpu.VMEM((CHUNK,), jnp.float32),
            pltpu.VMEM((CHUNK,), jnp.float32),
            pltpu.VMEM((CHUNK,), jnp.float32),
            pltpu.VMEM((CHUNK,), jnp.float32),
            pltpu.VMEM((8, LANES), jnp.float32),
            pltpu.VMEM((CHUNK,), jnp.float32),
            pltpu.SemaphoreType.DMA,
            pltpu.SemaphoreType.DMA,
        ],
    )
    return run(t_sorted.reshape(-1), img_flat, sx, sy, ex, ey,
               jnp.pad(scal, ((0, 2), (0, 0))))
